# Initial kernel scaffold; baseline (speedup 1.0000x reference)
import jax
import jax.numpy as jnp
from jax.experimental import pallas as pl


def kernel(x, edge_index, edge_attr, batch_idx, W_emb, b_emb, rgcn_w, rgcn_root, rgcn_b, mf_wl, mf_bl, mf_wr, W1, b1, W2, b2):
    raise NotImplementedError("write your pallas kernel here")


# speedup vs baseline: 2.6941x; 2.6941x over previous
import dataclasses
import functools
import jax
import jax.numpy as jnp
from jax import lax
from jax.experimental import pallas as pl
from jax.experimental.pallas import tpu as pltpu
from jax.experimental.pallas import tpu_sc as plsc

N = 10000
E = 320000
D_IN = 128
D_H = 128
D_OUT = 64
NUM_REL = 4
NUM_BLOCKS = 2
MAX_DEG = 10
NUM_GRAPHS = 64

NC = 2
NS = 16
NW = NC * NS
EPW = E // NW
KH = 80
KE = 80

_mesh = plsc.VectorSubcoreMesh(
    core_axis_name="c", subcore_axis_name="s", num_cores=NC, num_subcores=NS)

_cp = pltpu.CompilerParams()
if "needs_layout_passes" in pltpu.CompilerParams.__dataclass_fields__:
    _cp = dataclasses.replace(_cp, needs_layout_passes=False)


def _wid():
    return lax.axis_index("s") * NC + lax.axis_index("c")


def _hist_kernel(sidx_hbm, ones_hbm, zeros_hbm, out_hbm, idx_v, ones_v, acc,
                 sem):
    core = lax.axis_index("c")
    sub = lax.axis_index("s")
    wid = _wid()
    rows = (N * NUM_REL) // 10

    @pl.when(sub < 10)
    def _():
        pltpu.sync_copy(zeros_hbm.at[pl.ds(sub * rows, rows)],
                        acc.at[pl.ds(sub * rows, rows)])
    pltpu.sync_copy(ones_hbm, ones_v)
    plsc.subcore_barrier()

    @pl.loop(0, EPW // KH)
    def _(c):
        off = wid * EPW + c * KH
        pltpu.sync_copy(sidx_hbm.at[pl.ds(off, KH)], idx_v.at[0])
        pltpu.sync_copy(ones_v, acc.at[idx_v.at[0]], add=True)

    plsc.subcore_barrier()

    @pl.when(sub < 10)
    def _():
        pltpu.sync_copy(acc.at[pl.ds(sub * rows, rows)],
                        out_hbm.at[core].at[pl.ds(sub * rows, rows)])


@jax.jit
def _sc_hist(sidx):
    ones = jnp.ones((KH, 16), jnp.float32)
    zeros = jnp.zeros((N * NUM_REL, 16), jnp.float32)
    f = pl.kernel(
        _hist_kernel,
        out_type=jax.ShapeDtypeStruct((NC, N * NUM_REL, 16), jnp.float32),
        mesh=_mesh,
        compiler_params=_cp,
        scratch_types=[
            pltpu.VMEM((1, KH), jnp.int32),
            pltpu.VMEM((KH, 16), jnp.float32),
            pltpu.VMEM_SHARED((N * NUM_REL, 16), jnp.float32),
            pltpu.SemaphoreType.DMA,
        ],
    )
    part = f(sidx, ones, zeros)
    return part[0, :, 0] + part[1, :, 0]


def _sgather_kernel(sidx_hbm, inv_hbm, s_hbm, idx_v, sbuf_v, inv_v, sem):
    wid = _wid()
    pltpu.sync_copy(inv_hbm, inv_v)

    @pl.loop(0, EPW // KH)
    def _(c):
        off = wid * EPW + c * KH
        pltpu.sync_copy(sidx_hbm.at[pl.ds(off, KH)], idx_v.at[0])

        @pl.loop(0, KH, step=16)
        def _(kk):
            iv = idx_v[0, pl.ds(kk, 16)]
            sbuf_v[0, pl.ds(kk, 16)] = plsc.load_gather(inv_v, [iv])

        pltpu.sync_copy(sbuf_v.at[0], s_hbm.at[pl.ds(off, KH)])


@jax.jit
def _sc_sgather(sidx, inv):
    f = pl.kernel(
        _sgather_kernel,
        out_type=jax.ShapeDtypeStruct((E,), jnp.float32),
        mesh=_mesh,
        compiler_params=_cp,
        scratch_types=[
            pltpu.VMEM((1, KH), jnp.int32),
            pltpu.VMEM((1, KH), jnp.float32),
            pltpu.VMEM((N * NUM_REL,), jnp.float32),
            pltpu.SemaphoreType.DMA,
        ],
    )
    return f(sidx, inv)


def _rgcn_kernel(ytab_hbm, gidx_hbm, didx_hbm, s_hbm, zeros_hbm,
                 out_hbm, gidx_v, didx_v, s_v, rows_v, acc, sem1):
    core = lax.axis_index("c")
    sub = lax.axis_index("s")
    wid = _wid()
    rows = N // 10

    @pl.when(sub < 10)
    def _():
        pltpu.sync_copy(zeros_hbm.at[pl.ds(sub * rows, rows)],
                        acc.at[pl.ds(sub * rows, rows)])
    plsc.subcore_barrier()

    @pl.loop(0, EPW // KE)
    def _(c):
        off = wid * EPW + c * KE
        pltpu.sync_copy(gidx_hbm.at[pl.ds(off, KE)], gidx_v.at[0])
        pltpu.sync_copy(didx_hbm.at[pl.ds(off, KE)], didx_v.at[0])
        pltpu.sync_copy(s_hbm.at[pl.ds(off, KE)], s_v.at[0])
        pltpu.async_copy(ytab_hbm.at[gidx_v.at[0]], rows_v, sem1).wait()

        @pl.loop(0, KE, step=16)
        def _(kk):
            sv = s_v[0, pl.ds(kk, 16)]
            for j in range(16):
                row = rows_v.at[kk + j]
                sj = jnp.full((16,), sv[j], jnp.float32)
                for q in range(8):
                    sl = pl.ds(q * 16, 16)
                    row[sl] = row[sl] * sj

        pltpu.sync_copy(rows_v, acc.at[didx_v.at[0]], add=True)

    plsc.subcore_barrier()

    @pl.when(sub < 10)
    def _():
        pltpu.sync_copy(acc.at[pl.ds(sub * rows, rows)],
                        out_hbm.at[core].at[pl.ds(sub * rows, rows)])


@jax.jit
def _sc_rgcn(ytab, gidx, didx, s):
    zeros = jnp.zeros((N, D_H), jnp.float32)
    f = pl.kernel(
        _rgcn_kernel,
        out_type=jax.ShapeDtypeStruct((NC, N, D_H), jnp.float32),
        mesh=_mesh,
        compiler_params=_cp,
        scratch_types=[
            pltpu.VMEM((1, KE), jnp.int32),
            pltpu.VMEM((1, KE), jnp.int32),
            pltpu.VMEM((1, KE), jnp.float32),
            pltpu.VMEM((KE, D_H), jnp.float32),
            pltpu.VMEM_SHARED((N, D_H), jnp.float32),
            pltpu.SemaphoreType.DMA,
        ],
    )
    part = f(ytab, gidx, didx, s, zeros)
    return part[0] + part[1]


def _agg_kernel(htab_hbm, gidx_hbm, didx_hbm, zeros_hbm,
                out_hbm, gidx_v, didx_v, rows_v, acc, sem1):
    core = lax.axis_index("c")
    sub = lax.axis_index("s")
    wid = _wid()
    rows = N // 10

    @pl.when(sub < 10)
    def _():
        pltpu.sync_copy(zeros_hbm.at[pl.ds(sub * rows, rows)],
                        acc.at[pl.ds(sub * rows, rows)])
    plsc.subcore_barrier()

    @pl.loop(0, EPW // KE)
    def _(c):
        off = wid * EPW + c * KE
        pltpu.sync_copy(gidx_hbm.at[pl.ds(off, KE)], gidx_v.at[0])
        pltpu.sync_copy(didx_hbm.at[pl.ds(off, KE)], didx_v.at[0])
        pltpu.async_copy(htab_hbm.at[gidx_v.at[0]], rows_v, sem1).wait()
        pltpu.sync_copy(rows_v, acc.at[didx_v.at[0]], add=True)

    plsc.subcore_barrier()

    @pl.when(sub < 10)
    def _():
        pltpu.sync_copy(acc.at[pl.ds(sub * rows, rows)],
                        out_hbm.at[core].at[pl.ds(sub * rows, rows)])


@jax.jit
def _sc_agg(htab, gidx, didx):
    zeros = jnp.zeros((N, D_H), jnp.float32)
    f = pl.kernel(
        _agg_kernel,
        out_type=jax.ShapeDtypeStruct((NC, N, D_H), jnp.float32),
        mesh=_mesh,
        compiler_params=_cp,
        scratch_types=[
            pltpu.VMEM((1, KE), jnp.int32),
            pltpu.VMEM((1, KE), jnp.int32),
            pltpu.VMEM((KE, D_H), jnp.float32),
            pltpu.VMEM_SHARED((N, D_H), jnp.float32),
            pltpu.SemaphoreType.DMA,
        ],
    )
    part = f(htab, gidx, didx, zeros)
    return part[0] + part[1]


def kernel(x, edge_index, edge_attr, batch_idx, W_emb, b_emb, rgcn_w,
           rgcn_root, rgcn_b, mf_wl, mf_bl, mf_wr, W1, b1, W2, b2):
    src, dst = edge_index[0], edge_index[1]
    gidx = src * NUM_REL + edge_attr
    sidx = dst * NUM_REL + edge_attr
    didx = dst

    cnt = jnp.zeros((N * NUM_REL,), jnp.float32).at[sidx].add(1.0)
    cnt2 = cnt.reshape(N, NUM_REL)
    deg = jnp.clip(jnp.sum(cnt2, axis=1), 0, MAX_DEG).astype(jnp.int32)
    inv = 1.0 / jnp.clip(cnt, 1.0, None)
    s_edge = inv[sidx]

    h = x @ W_emb + b_emb
    for l in range(NUM_BLOCKS):
        hr = jax.nn.relu(h)
        base = hr @ rgcn_root[l] + rgcn_b[l]
        wcat = jnp.transpose(rgcn_w[l], (1, 0, 2)).reshape(D_H, NUM_REL * D_H)
        ytab = (hr @ wcat).reshape(N * NUM_REL, D_H)
        msg = _sc_rgcn(ytab, gidx, didx, s_edge)
        hm = jax.nn.relu(base + msg)
        agg = _sc_agg(hm, src, didx)
        z = jnp.concatenate([agg, hm], axis=1)
        wl = jnp.concatenate([mf_wl[l], mf_wr[l]], axis=1)
        out = jnp.zeros((N, D_H), jnp.float32)
        for d in range(MAX_DEG + 1):
            r = z @ wl[d] + mf_bl[l, d]
            out = jnp.where((deg == d)[:, None], r, out)
        h = out

    onehot = (batch_idx[:, None] == jnp.arange(NUM_GRAPHS)[None, :])
    pooled = onehot.astype(jnp.float32).T @ h
    return jax.nn.relu(pooled @ W1 + b1) @ W2 + b2


# speedup vs baseline: 6.7957x; 2.5224x over previous
import dataclasses
import functools
import jax
import jax.numpy as jnp
from jax import lax
from jax.experimental import pallas as pl
from jax.experimental.pallas import tpu as pltpu
from jax.experimental.pallas import tpu_sc as plsc

N = 10000
E = 320000
D_IN = 128
D_H = 128
D_OUT = 64
NUM_REL = 4
NUM_BLOCKS = 2
MAX_DEG = 10
NUM_GRAPHS = 64

NC = 2
NS = 16
NW = NC * NS
EPW = E // NW
KH = 80
KE = 80

_mesh = plsc.VectorSubcoreMesh(
    core_axis_name="c", subcore_axis_name="s", num_cores=NC, num_subcores=NS)

_cp = pltpu.CompilerParams()
if "needs_layout_passes" in pltpu.CompilerParams.__dataclass_fields__:
    _cp = dataclasses.replace(_cp, needs_layout_passes=False)


def _wid():
    return lax.axis_index("s") * NC + lax.axis_index("c")


def _hist_kernel(sidx_hbm, zeros_hbm, out_hbm, idx_v, hist_v, sem):
    wid = _wid()
    ones = jnp.ones((16,), jnp.float32)
    pltpu.sync_copy(zeros_hbm, hist_v)

    @pl.loop(0, EPW // KH)
    def _(c):
        off = wid * EPW + c * KH
        pltpu.sync_copy(sidx_hbm.at[pl.ds(off, KH)], idx_v.at[0])

        @pl.loop(0, KH, step=16)
        def _(kk):
            iv = idx_v[0, pl.ds(kk, 16)]
            plsc.addupdate_scatter(hist_v, [iv], ones)

    pltpu.sync_copy(hist_v, out_hbm.at[wid])


@jax.jit
def _sc_hist(sidx):
    zeros = jnp.zeros((N * NUM_REL,), jnp.float32)
    f = pl.kernel(
        _hist_kernel,
        out_type=jax.ShapeDtypeStruct((NW, N * NUM_REL), jnp.float32),
        mesh=_mesh,
        compiler_params=_cp,
        scratch_types=[
            pltpu.VMEM((1, KH), jnp.int32),
            pltpu.VMEM((N * NUM_REL,), jnp.float32),
            pltpu.SemaphoreType.DMA,
        ],
    )
    return jnp.sum(f(sidx, zeros), axis=0)


def _rgcn_kernel(ytab_hbm, gidx_hbm, didx_hbm, sidx_hbm, inv_hbm, zeros_hbm,
                 out_hbm, gidx_v, didx_v, sidx_v, rows_v, inv_v, acc, sem1):
    core = lax.axis_index("c")
    sub = lax.axis_index("s")
    wid = _wid()
    rows = N // 10

    @pl.when(sub < 10)
    def _():
        pltpu.sync_copy(zeros_hbm.at[pl.ds(sub * rows, rows)],
                        acc.at[pl.ds(sub * rows, rows)])
    pltpu.sync_copy(inv_hbm, inv_v)
    plsc.subcore_barrier()

    @pl.loop(0, EPW // KE)
    def _(c):
        off = wid * EPW + c * KE
        pltpu.sync_copy(gidx_hbm.at[pl.ds(off, KE)], gidx_v.at[0])
        pltpu.sync_copy(didx_hbm.at[pl.ds(off, KE)], didx_v.at[0])
        pltpu.sync_copy(sidx_hbm.at[pl.ds(off, KE)], sidx_v.at[0])
        pltpu.async_copy(ytab_hbm.at[gidx_v.at[0]], rows_v, sem1).wait()

        @pl.loop(0, KE, step=16)
        def _(kk):
            iv = sidx_v[0, pl.ds(kk, 16)]
            sv = plsc.load_gather(inv_v, [iv])
            for j in range(16):
                row = rows_v.at[kk + j]
                sj = jnp.full((16,), sv[j], jnp.float32)
                for q in range(8):
                    sl = pl.ds(q * 16, 16)
                    row[sl] = row[sl] * sj

        pltpu.sync_copy(rows_v, acc.at[didx_v.at[0]], add=True)

    plsc.subcore_barrier()

    @pl.when(sub < 10)
    def _():
        pltpu.sync_copy(acc.at[pl.ds(sub * rows, rows)],
                        out_hbm.at[core].at[pl.ds(sub * rows, rows)])


@jax.jit
def _sc_rgcn(ytab, gidx, didx, sidx, inv):
    zeros = jnp.zeros((N, D_H), jnp.float32)
    f = pl.kernel(
        _rgcn_kernel,
        out_type=jax.ShapeDtypeStruct((NC, N, D_H), jnp.float32),
        mesh=_mesh,
        compiler_params=_cp,
        scratch_types=[
            pltpu.VMEM((1, KE), jnp.int32),
            pltpu.VMEM((1, KE), jnp.int32),
            pltpu.VMEM((1, KE), jnp.int32),
            pltpu.VMEM((KE, D_H), jnp.float32),
            pltpu.VMEM((N * NUM_REL,), jnp.float32),
            pltpu.VMEM_SHARED((N, D_H), jnp.float32),
            pltpu.SemaphoreType.DMA,
        ],
    )
    part = f(ytab, gidx, didx, sidx, inv, zeros)
    return part[0] + part[1]


def _agg_kernel(htab_hbm, gidx_hbm, didx_hbm, zeros_hbm,
                out_hbm, gidx_v, didx_v, rows_v, acc, sem1):
    core = lax.axis_index("c")
    sub = lax.axis_index("s")
    wid = _wid()
    rows = N // 10

    @pl.when(sub < 10)
    def _():
        pltpu.sync_copy(zeros_hbm.at[pl.ds(sub * rows, rows)],
                        acc.at[pl.ds(sub * rows, rows)])
    plsc.subcore_barrier()

    @pl.loop(0, EPW // KE)
    def _(c):
        off = wid * EPW + c * KE
        pltpu.sync_copy(gidx_hbm.at[pl.ds(off, KE)], gidx_v.at[0])
        pltpu.sync_copy(didx_hbm.at[pl.ds(off, KE)], didx_v.at[0])
        pltpu.async_copy(htab_hbm.at[gidx_v.at[0]], rows_v, sem1).wait()
        pltpu.sync_copy(rows_v, acc.at[didx_v.at[0]], add=True)

    plsc.subcore_barrier()

    @pl.when(sub < 10)
    def _():
        pltpu.sync_copy(acc.at[pl.ds(sub * rows, rows)],
                        out_hbm.at[core].at[pl.ds(sub * rows, rows)])


@jax.jit
def _sc_agg(htab, gidx, didx):
    zeros = jnp.zeros((N, D_H), jnp.float32)
    f = pl.kernel(
        _agg_kernel,
        out_type=jax.ShapeDtypeStruct((NC, N, D_H), jnp.float32),
        mesh=_mesh,
        compiler_params=_cp,
        scratch_types=[
            pltpu.VMEM((1, KE), jnp.int32),
            pltpu.VMEM((1, KE), jnp.int32),
            pltpu.VMEM((KE, D_H), jnp.float32),
            pltpu.VMEM_SHARED((N, D_H), jnp.float32),
            pltpu.SemaphoreType.DMA,
        ],
    )
    part = f(htab, gidx, didx, zeros)
    return part[0] + part[1]


def kernel(x, edge_index, edge_attr, batch_idx, W_emb, b_emb, rgcn_w,
           rgcn_root, rgcn_b, mf_wl, mf_bl, mf_wr, W1, b1, W2, b2):
    src, dst = edge_index[0], edge_index[1]
    gidx = src * NUM_REL + edge_attr
    sidx = dst * NUM_REL + edge_attr
    didx = dst

    cnt = _sc_hist(sidx)
    cnt2 = cnt.reshape(N, NUM_REL)
    deg = jnp.clip(jnp.sum(cnt2, axis=1), 0, MAX_DEG).astype(jnp.int32)
    inv = 1.0 / jnp.clip(cnt, 1.0, None)

    h = x @ W_emb + b_emb
    for l in range(NUM_BLOCKS):
        hr = jax.nn.relu(h)
        base = hr @ rgcn_root[l] + rgcn_b[l]
        wcat = jnp.transpose(rgcn_w[l], (1, 0, 2)).reshape(D_H, NUM_REL * D_H)
        ytab = (hr @ wcat).reshape(N * NUM_REL, D_H)
        msg = _sc_rgcn(ytab, gidx, didx, sidx, inv)
        hm = jax.nn.relu(base + msg)
        agg = _sc_agg(hm, src, didx)
        z = jnp.concatenate([agg, hm], axis=1)
        wl = jnp.concatenate([mf_wl[l], mf_wr[l]], axis=1)
        out = jnp.zeros((N, D_H), jnp.float32)
        for d in range(MAX_DEG + 1):
            r = z @ wl[d] + mf_bl[l, d]
            out = jnp.where((deg == d)[:, None], r, out)
        h = out

    onehot = (batch_idx[:, None] == jnp.arange(NUM_GRAPHS)[None, :])
    pooled = onehot.astype(jnp.float32).T @ h
    return jax.nn.relu(pooled @ W1 + b1) @ W2 + b2
